# baseline (speedup 1.0000x reference)
import functools

import jax
import jax.numpy as jnp
from jax import lax
from jax.experimental import pallas as pl
from jax.experimental.pallas import tpu as pltpu
from jax.experimental.pallas import tpu_sc as plsc

N = 10000
E = 320000
D = 128
NMOL = 4096
DEPTH = 3

BLK = 512
BLK_F = 256
S_ROUND = 10240
NW = 32
CHUNK = 128
DUMP = 128

RT = S_ROUND // NW
RT_P = NMOL // NW
CAP = 96
CAP_P = 4


@functools.cache
def _make_sc_agg(rows_sc: int, cap: int):
    rt = rows_sc // 16
    own = (rows_sc - DUMP) // 16
    mesh = plsc.VectorSubcoreMesh(core_axis_name="c", subcore_axis_name="s",
                                  num_cores=2, num_subcores=16)

    @functools.partial(
        pl.kernel,
        mesh=mesh,
        out_type=jax.ShapeDtypeStruct((2 * (rows_sc - DUMP), D), jnp.float32),
        scratch_types=[
            pltpu.VMEM((cap, CHUNK), jnp.int32),
            pltpu.VMEM((cap, CHUNK), jnp.int32),
            pltpu.VMEM((CHUNK,), jnp.int32),
            pltpu.VMEM((CHUNK, D), jnp.float32),
            pltpu.VMEM_SHARED((rows_sc, D), jnp.float32),
            pltpu.SemaphoreType.DMA,
        ],
    )
    def sc_agg(feats_hbm, srcs_hbm, dsts_hbm, cnts_hbm, zeros_hbm, out_hbm,
               src_v, dst_v, cnt_v, buf, agg_s, gs):
        cid = lax.axis_index("c")
        sid = lax.axis_index("s")
        w = cid * 16 + sid
        pltpu.sync_copy(srcs_hbm.at[w], src_v)
        pltpu.sync_copy(dsts_hbm.at[w], dst_v)
        pltpu.sync_copy(cnts_hbm.at[w], cnt_v)
        pltpu.sync_copy(zeros_hbm.at[pl.ds(sid * rt, rt)],
                        agg_s.at[pl.ds(sid * rt, rt)])
        plsc.subcore_barrier()

        def body(j, carry):
            pltpu.async_copy(feats_hbm.at[src_v.at[j]], buf, gs).wait()
            pltpu.sync_copy(buf, agg_s.at[dst_v.at[j]], add=True)
            return carry

        cnt = cnt_v[pl.ds(0, 16)][0]
        lax.fori_loop(0, cnt, body, 0)
        plsc.subcore_barrier()
        pltpu.sync_copy(agg_s.at[pl.ds(sid * own, own)],
                        out_hbm.at[pl.ds(cid * (rows_sc - DUMP) + sid * own,
                                         own)])

    return sc_agg


def _route(src, dst, rows_per_tile, rows_sc, cap):
    bucket = dst // rows_per_tile
    perm = jnp.argsort(bucket, stable=True)
    src_s = src[perm]
    dst_s = dst[perm]
    n_w = jnp.bincount(bucket, length=NW)
    off = jnp.cumsum(n_w) - n_w
    cnts = -(-n_w // CHUNK)
    pos = (off[:, None, None]
           + (jnp.arange(cap) * CHUNK)[None, :, None]
           + jnp.arange(CHUNK)[None, None, :])
    valid = pos < (off + n_w)[:, None, None]
    posc = jnp.clip(pos, 0, src_s.shape[0] - 1).astype(jnp.int32)
    srcs = jnp.where(valid, src_s[posc], 0).astype(jnp.int32)
    sc_base = (jnp.arange(NW) // 16 * (rows_sc - DUMP))[:, None, None]
    dump_rel = (rows_sc - DUMP) + (pos % DUMP)
    dsts = jnp.where(valid, dst_s[posc] - sc_base,
                     dump_rel).astype(jnp.int32)
    cnts_b = jnp.broadcast_to(cnts.astype(jnp.int32)[:, None], (NW, CHUNK))
    return srcs, dsts, cnts_b


def _tc_h0(x_p, W_in):
    def body(x_ref, wi_ref, h0_ref):
        h0_ref[...] = jnp.maximum(
            jnp.dot(x_ref[...], wi_ref[...],
                    preferred_element_type=jnp.float32), 0.0)

    return pl.pallas_call(
        body,
        grid=(S_ROUND // BLK,),
        in_specs=[
            pl.BlockSpec((BLK, D), lambda i: (i, 0)),
            pl.BlockSpec((D, D), lambda i: (0, 0)),
        ],
        out_specs=pl.BlockSpec((BLK, D), lambda i: (i, 0)),
        out_shape=jax.ShapeDtypeStruct((S_ROUND, D), jnp.float32),
    )(x_p, W_in)


def _tc_round(h0, q, W_h):
    def body(h0_ref, q_ref, wh_ref, h_ref):
        h_ref[...] = jnp.maximum(
            h0_ref[...]
            + jnp.dot(q_ref[...], wh_ref[...],
                      preferred_element_type=jnp.float32), 0.0)

    return pl.pallas_call(
        body,
        grid=(S_ROUND // BLK,),
        in_specs=[
            pl.BlockSpec((BLK, D), lambda i: (i, 0)),
            pl.BlockSpec((BLK, D), lambda i: (i, 0)),
            pl.BlockSpec((D, D), lambda i: (0, 0)),
        ],
        out_specs=pl.BlockSpec((BLK, D), lambda i: (i, 0)),
        out_shape=jax.ShapeDtypeStruct((S_ROUND, D), jnp.float32),
    )(h0, q, W_h)


def _tc_final(m, W_ffn1, b_ffn1, W_out_p, b_out_p):
    def body(m_ref, w1_ref, b1_ref, wo_ref, bo_ref, out_ref):
        z = jnp.maximum(
            jnp.dot(m_ref[...], w1_ref[...],
                    preferred_element_type=jnp.float32) + b1_ref[...], 0.0)
        out_ref[...] = (
            jnp.dot(z, wo_ref[...], preferred_element_type=jnp.float32)
            + bo_ref[...])

    return pl.pallas_call(
        body,
        grid=(NMOL // BLK_F,),
        in_specs=[
            pl.BlockSpec((BLK_F, D), lambda i: (i, 0)),
            pl.BlockSpec((D, D), lambda i: (0, 0)),
            pl.BlockSpec((1, D), lambda i: (0, 0)),
            pl.BlockSpec((D, D), lambda i: (0, 0)),
            pl.BlockSpec((1, D), lambda i: (0, 0)),
        ],
        out_specs=pl.BlockSpec((BLK_F, D), lambda i: (i, 0)),
        out_shape=jax.ShapeDtypeStruct((NMOL, D), jnp.float32),
    )(m, W_ffn1, b_ffn1, W_out_p, b_out_p)


def kernel(x, edge_index, mol_ids, W_in, W_h, W_ffn1, b_ffn1, W_out, b_out):
    src = edge_index[0].astype(jnp.int32)
    dst = edge_index[1].astype(jnp.int32)

    rows_sc_r = S_ROUND // 2 + DUMP
    rows_sc_p = NMOL // 2 + DUMP
    srcs_r, dsts_r, cnts_r = _route(src, dst, RT, rows_sc_r, CAP)
    srcs_p, dsts_p, cnts_p = _route(
        jnp.arange(N, dtype=jnp.int32), mol_ids.astype(jnp.int32),
        RT_P, rows_sc_p, CAP_P)
    zeros = jnp.zeros((rows_sc_r, D), jnp.float32)
    x_p = jnp.pad(x, ((0, S_ROUND - N), (0, 0)))

    h = _tc_h0(x_p, W_in)
    h0 = h
    sc_round = _make_sc_agg(rows_sc_r, CAP)
    for _ in range(DEPTH):
        q = sc_round(h, srcs_r, dsts_r, cnts_r, zeros)
        h = _tc_round(h0, q, W_h)

    sc_pool = _make_sc_agg(rows_sc_p, CAP_P)
    m = sc_pool(h, srcs_p, dsts_p, cnts_p, zeros)

    W_out_p = jnp.pad(W_out, ((0, 0), (0, D - W_out.shape[1])))
    b_out_p = jnp.pad(b_out, (0, D - b_out.shape[0])).reshape(1, D)
    out_full = _tc_final(m, W_ffn1, b_ffn1.reshape(1, D), W_out_p, b_out_p)
    return out_full[:, :W_out.shape[1]]
